# baseline (speedup 1.0000x reference)
import jax
import jax.numpy as jnp
from jax.experimental import pallas as pl
from jax.experimental.pallas import tpu as pltpu

B, S, H, E, K = 4, 2048, 1024, 8, 2
CAP = (S * K) // E
CH = 1024
NCH = S // CH
TRI = 256


def _body(hs_ref, wt_ref, probs_ref, disp_ref, comb_ref, aux_ref, z_ref,
          ohb_s, wb_s, hist_s):
    ph = pl.program_id(1)
    c = pl.program_id(2)

    @pl.when(ph == 0)
    def _phase0():
        hs = hs_ref[0]
        logits = jnp.dot(hs, wt_ref[...],
                         preferred_element_type=jnp.float32)
        m = jnp.max(logits, axis=-1, keepdims=True)
        el = jnp.exp(logits - m)
        sel = jnp.sum(el, axis=-1, keepdims=True)
        probs = el / sel
        probs_ref[0] = probs

        lse = m + jnp.log(sel)
        z_ref[...] = jnp.sum(lse * lse).reshape(1, 1, 1, 1)
        aux_ref[...] = jnp.sum(probs * probs).reshape(1, 1, 1, 1)

        eidx = jax.lax.broadcasted_iota(jnp.int32, (CH, E), 1)
        m1 = jnp.max(probs, axis=-1, keepdims=True)
        i1 = jnp.min(jnp.where(probs == m1, eidx, E), axis=-1, keepdims=True)
        p2 = jnp.where(eidx == i1, -1.0, probs)
        m2 = jnp.max(p2, axis=-1, keepdims=True)
        i2 = jnp.min(jnp.where(p2 == m2, eidx, E), axis=-1, keepdims=True)

        wsum = m1 + m2
        oh1 = eidx == i1
        oh2 = eidx == i2
        ohb = jnp.concatenate(
            [oh1.astype(jnp.bfloat16), oh2.astype(jnp.bfloat16)], axis=1)
        ohb_s[pl.ds(c * CH, CH), :] = ohb
        wb = jnp.where(oh1, m1 / wsum, 0.0) + jnp.where(oh2, m2 / wsum, 0.0)
        wb_s[pl.ds(c * CH, CH), :] = wb
        hist_s[pl.ds(c, 1), :] = jnp.sum(
            ohb.astype(jnp.float32), axis=0, keepdims=True)

    @pl.when(ph == 1)
    def _phase1():
        hist = hist_s[...]
        cidx = jax.lax.broadcasted_iota(jnp.int32, (NCH, 2 * E), 0)
        carry = jnp.sum(jnp.where(cidx < c, hist, 0.0), axis=0,
                        keepdims=True)
        c0tot = jnp.sum(hist[:, :E], axis=0, keepdims=True)

        ohb = ohb_s[pl.ds(c * CH, CH), :]
        r_i = jax.lax.broadcasted_iota(jnp.int32, (TRI, TRI), 0)
        c_i = jax.lax.broadcasted_iota(jnp.int32, (TRI, TRI), 1)
        ls = (r_i > c_i).astype(jnp.bfloat16)
        outs = []
        for t in range(CH // TRI):
            blk = ohb[t * TRI:(t + 1) * TRI]
            excl = jnp.dot(ls, blk, preferred_element_type=jnp.float32)
            outs.append(excl + carry)
            carry = carry + jnp.sum(blk.astype(jnp.float32), axis=0,
                                    keepdims=True)
        rank = jnp.concatenate(outs, axis=0)
        r1 = rank[:, :E]
        r2 = rank[:, E:] + c0tot

        ohf = ohb.astype(jnp.float32)
        a1 = ohf[:, :E] * (r1 < CAP).astype(jnp.float32)
        a2 = ohf[:, E:] * (r2 < CAP).astype(jnp.float32)
        disp_ref[0] = a1 + a2
        comb_ref[0] = (a1 + a2) * wb_s[pl.ds(c * CH, CH), :]


@jax.jit
def kernel(hidden_states, W):
    wt = W.T
    last = NCH - 1
    probs, disp, comb, aux, z = pl.pallas_call(
        _body,
        grid=(B, 2, NCH),
        in_specs=[
            pl.BlockSpec((1, CH, H),
                         lambda b, ph, c: (b, jnp.where(ph == 0, c, last), 0)),
            pl.BlockSpec((H, E), lambda b, ph, c: (0, 0)),
        ],
        out_specs=[
            pl.BlockSpec((1, CH, E),
                         lambda b, ph, c: (b, jnp.where(ph == 0, c, last), 0)),
            pl.BlockSpec((1, CH, E),
                         lambda b, ph, c: (b, jnp.where(ph == 1, c, 0), 0)),
            pl.BlockSpec((1, CH, E),
                         lambda b, ph, c: (b, jnp.where(ph == 1, c, 0), 0)),
            pl.BlockSpec((1, 1, 1, 1),
                         lambda b, ph, c: (b, jnp.where(ph == 0, c, last), 0, 0)),
            pl.BlockSpec((1, 1, 1, 1),
                         lambda b, ph, c: (b, jnp.where(ph == 0, c, last), 0, 0)),
        ],
        out_shape=[
            jax.ShapeDtypeStruct((B, S, E), jnp.float32),
            jax.ShapeDtypeStruct((B, S, E), jnp.float32),
            jax.ShapeDtypeStruct((B, S, E), jnp.float32),
            jax.ShapeDtypeStruct((B, NCH, 1, 1), jnp.float32),
            jax.ShapeDtypeStruct((B, NCH, 1, 1), jnp.float32),
        ],
        scratch_shapes=[
            pltpu.VMEM((S, 2 * E), jnp.bfloat16),
            pltpu.VMEM((S, E), jnp.float32),
            pltpu.VMEM((NCH, 2 * E), jnp.float32),
        ],
    )(hidden_states, wt)
    aux_loss = (jnp.sum(aux) / (B * S)) * E
    z_loss = jnp.sum(z) / (B * S)
    return (disp, comb, probs, aux_loss.reshape(()), z_loss.reshape(()))


# speedup vs baseline: 1.1789x; 1.1789x over previous
import functools

import jax
import jax.numpy as jnp
from jax.experimental import pallas as pl

B, S, H, E, K = 4, 2048, 1024, 8, 2
CAP = (S * K) // E


def _router_body(hs_ref, wt_ref, disp_ref, comb_ref, probs_ref, aux_ref, z_ref):
    hs = hs_ref[0]
    wt = wt_ref[...]
    logits = jnp.dot(hs, wt, preferred_element_type=jnp.float32)

    m = jnp.max(logits, axis=-1, keepdims=True)
    el = jnp.exp(logits - m)
    sel = jnp.sum(el, axis=-1, keepdims=True)
    probs = el / sel
    probs_ref[0] = probs

    lse = m + jnp.log(sel)
    z_ref[...] = jnp.sum(lse * lse).reshape(1, 1, 1)
    aux_ref[...] = jnp.sum(probs * probs).reshape(1, 1, 1)

    eidx = jax.lax.broadcasted_iota(jnp.int32, (S, E), 1)
    m1 = jnp.max(probs, axis=-1, keepdims=True)
    i1 = jnp.min(jnp.where(probs == m1, eidx, E), axis=-1, keepdims=True)
    p2 = jnp.where(eidx == i1, -1.0, probs)
    m2 = jnp.max(p2, axis=-1, keepdims=True)
    i2 = jnp.min(jnp.where(p2 == m2, eidx, E), axis=-1, keepdims=True)

    wsum = m1 + m2
    w1 = m1 / wsum
    w2 = m2 / wsum

    oh1 = eidx == i1
    oh2 = eidx == i2

    ohb = jnp.concatenate(
        [oh1.astype(jnp.bfloat16), oh2.astype(jnp.bfloat16)], axis=1)
    CH = 256
    r_idx = jax.lax.broadcasted_iota(jnp.int32, (CH, CH), 0)
    c_idx = jax.lax.broadcasted_iota(jnp.int32, (CH, CH), 1)
    ls = (r_idx > c_idx).astype(jnp.bfloat16)
    carry = jnp.zeros((1, 2 * E), jnp.float32)
    outs = []
    for c in range(S // CH):
        blk = ohb[c * CH:(c + 1) * CH]
        excl = jnp.dot(ls, blk, preferred_element_type=jnp.float32)
        outs.append(excl + carry)
        carry = carry + jnp.sum(blk.astype(jnp.float32), axis=0, keepdims=True)
    rank = jnp.concatenate(outs, axis=0)
    r1 = rank[:, :E]
    r2 = rank[:, E:] + carry[:, :E]

    a1 = (oh1 & (r1 < CAP)).astype(jnp.float32)
    a2 = (oh2 & (r2 < CAP)).astype(jnp.float32)
    disp_ref[0] = a1 + a2
    comb_ref[0] = a1 * w1 + a2 * w2


@functools.partial(jax.jit, static_argnames=())
def kernel(hidden_states, W):
    wt = W.T
    disp, comb, probs, aux, z = pl.pallas_call(
        _router_body,
        grid=(B,),
        in_specs=[
            pl.BlockSpec((1, S, H), lambda b: (b, 0, 0)),
            pl.BlockSpec((H, E), lambda b: (0, 0)),
        ],
        out_specs=[
            pl.BlockSpec((1, S, E), lambda b: (b, 0, 0)),
            pl.BlockSpec((1, S, E), lambda b: (b, 0, 0)),
            pl.BlockSpec((1, S, E), lambda b: (b, 0, 0)),
            pl.BlockSpec((1, 1, 1), lambda b: (b, 0, 0)),
            pl.BlockSpec((1, 1, 1), lambda b: (b, 0, 0)),
        ],
        out_shape=[
            jax.ShapeDtypeStruct((B, S, E), jnp.float32),
            jax.ShapeDtypeStruct((B, S, E), jnp.float32),
            jax.ShapeDtypeStruct((B, S, E), jnp.float32),
            jax.ShapeDtypeStruct((B, 1, 1), jnp.float32),
            jax.ShapeDtypeStruct((B, 1, 1), jnp.float32),
        ],
    )(hidden_states, wt)
    aux_loss = (jnp.sum(aux) / (B * S)) * E
    z_loss = jnp.sum(z) / (B * S)
    return (disp, comb, probs, aux_loss.reshape(()), z_loss.reshape(()))


# speedup vs baseline: 1.2532x; 1.0631x over previous
import functools

import jax
import jax.numpy as jnp
from jax.experimental import pallas as pl

B, S, H, E, K = 4, 2048, 1024, 8, 2
CAP = (S * K) // E


def _router_body(hs_ref, wt_ref, disp_ref, comb_ref, probs_ref, aux_ref, z_ref):
    hs = hs_ref[0]
    wt = wt_ref[...]
    logits = jnp.dot(hs, wt, preferred_element_type=jnp.float32)

    m = jnp.max(logits, axis=-1, keepdims=True)
    el = jnp.exp(logits - m)
    sel = jnp.sum(el, axis=-1, keepdims=True)
    probs = el / sel
    probs_ref[0] = probs

    lse = m + jnp.log(sel)
    z_ref[...] = jnp.sum(lse * lse).reshape(1, 1, 1)
    aux_ref[...] = jnp.sum(probs * probs).reshape(1, 1, 1)

    eidx = jax.lax.broadcasted_iota(jnp.int32, (S, E), 1)
    m1 = jnp.max(probs, axis=-1, keepdims=True)
    i1 = jnp.min(jnp.where(probs == m1, eidx, E), axis=-1, keepdims=True)
    p2 = jnp.where(eidx == i1, -1.0, probs)
    m2 = jnp.max(p2, axis=-1, keepdims=True)
    i2 = jnp.min(jnp.where(p2 == m2, eidx, E), axis=-1, keepdims=True)

    wsum = m1 + m2
    w1 = m1 / wsum
    w2 = m2 / wsum

    oh1 = eidx == i1
    oh2 = eidx == i2

    ohf2 = jnp.concatenate(
        [oh1.astype(jnp.float32), oh2.astype(jnp.float32)], axis=1)
    x = ohf2
    d = 1
    while d < S:
        x = x + jnp.concatenate(
            [jnp.zeros((d, 2 * E), jnp.float32), x[:S - d]], axis=0)
        d *= 2
    rank = x - ohf2
    r1 = rank[:, :E]
    r2 = rank[:, E:] + x[S - 1:, :E]

    a1 = (oh1 & (r1 < CAP)).astype(jnp.float32)
    a2 = (oh2 & (r2 < CAP)).astype(jnp.float32)
    disp_ref[0] = a1 + a2
    comb_ref[0] = a1 * w1 + a2 * w2


@functools.partial(jax.jit, static_argnames=())
def kernel(hidden_states, W):
    wt = W.T
    disp, comb, probs, aux, z = pl.pallas_call(
        _router_body,
        grid=(B,),
        in_specs=[
            pl.BlockSpec((1, S, H), lambda b: (b, 0, 0)),
            pl.BlockSpec((H, E), lambda b: (0, 0)),
        ],
        out_specs=[
            pl.BlockSpec((1, S, E), lambda b: (b, 0, 0)),
            pl.BlockSpec((1, S, E), lambda b: (b, 0, 0)),
            pl.BlockSpec((1, S, E), lambda b: (b, 0, 0)),
            pl.BlockSpec((1, 1, 1), lambda b: (b, 0, 0)),
            pl.BlockSpec((1, 1, 1), lambda b: (b, 0, 0)),
        ],
        out_shape=[
            jax.ShapeDtypeStruct((B, S, E), jnp.float32),
            jax.ShapeDtypeStruct((B, S, E), jnp.float32),
            jax.ShapeDtypeStruct((B, S, E), jnp.float32),
            jax.ShapeDtypeStruct((B, 1, 1), jnp.float32),
            jax.ShapeDtypeStruct((B, 1, 1), jnp.float32),
        ],
    )(hidden_states, wt)
    aux_loss = (jnp.sum(aux) / (B * S)) * E
    z_loss = jnp.sum(z) / (B * S)
    return (disp, comb, probs, aux_loss.reshape(()), z_loss.reshape(()))
